# baseline (speedup 1.0000x reference)
import functools

import jax
import jax.numpy as jnp
from jax import lax
from jax.experimental import pallas as pl
from jax.experimental.pallas import tpu as pltpu
from jax.experimental.pallas import tpu_sc as plsc

NC = 2
NS = 16
L = 16
NW = NC * NS

BATCH = 16384
EMB = 64
B_PER_W = BATCH // NW
CHUNK = 128
NCHUNK = B_PER_W // CHUNK
GROUPS = B_PER_W // L


def _cf_body(user_ids_hbm, item_ids_hbm, user_table_hbm, item_table_hbm,
             item_bias_hbm, out_hbm,
             uidx_v, iidx_v, urows_v, irows_v, bias_v, out_v, sem):
    wid = lax.axis_index("s") * NC + lax.axis_index("c")
    base = wid * B_PER_W

    for c in range(NCHUNK):
        pltpu.sync_copy(user_ids_hbm.at[pl.ds(base + c * CHUNK, CHUNK)],
                        uidx_v.at[c])
        pltpu.sync_copy(item_ids_hbm.at[pl.ds(base + c * CHUNK, CHUNK)],
                        iidx_v.at[c])

    copies = []
    for c in range(NCHUNK):
        copies.append(pltpu.async_copy(
            user_table_hbm.at[uidx_v.at[c]],
            urows_v.at[pl.ds(c * CHUNK, CHUNK), :], sem))
        copies.append(pltpu.async_copy(
            item_table_hbm.at[iidx_v.at[c]],
            irows_v.at[pl.ds(c * CHUNK, CHUNK), :], sem))
        copies.append(pltpu.async_copy(
            item_bias_hbm.at[iidx_v.at[c]],
            bias_v.at[pl.ds(c * CHUNK, CHUNK)], sem))
    for cp in copies:
        cp.wait()

    lanes = lax.iota(jnp.int32, L)

    def group(g, _):
        row16 = g * L + lanes
        acc = bias_v[pl.ds(g * L, L)]
        for j in range(EMB):
            colj = jnp.full((L,), j, jnp.int32)
            u = plsc.load_gather(urows_v, [row16, colj])
            v = plsc.load_gather(irows_v, [row16, colj])
            acc = acc + u * v
        out_v[pl.ds(g * L, L)] = acc
        return 0

    lax.fori_loop(0, GROUPS, group, 0)

    pltpu.sync_copy(out_v, out_hbm.at[pl.ds(base, B_PER_W)])


@jax.jit
def kernel(user_ids, item_ids, user_table, item_table, item_bias):
    mesh = plsc.VectorSubcoreMesh(core_axis_name="c", subcore_axis_name="s")
    run = pl.kernel(
        _cf_body,
        out_type=jax.ShapeDtypeStruct((BATCH,), jnp.float32),
        mesh=mesh,
        scratch_types=[
            pltpu.VMEM((NCHUNK, CHUNK), jnp.int32),
            pltpu.VMEM((NCHUNK, CHUNK), jnp.int32),
            pltpu.VMEM((B_PER_W, EMB), jnp.float32),
            pltpu.VMEM((B_PER_W, EMB), jnp.float32),
            pltpu.VMEM((B_PER_W,), jnp.float32),
            pltpu.VMEM((B_PER_W,), jnp.float32),
            pltpu.SemaphoreType.DMA,
        ],
        compiler_params=pltpu.CompilerParams(needs_layout_passes=False,
                                             use_tc_tiling_on_sc=False),
        name="cf_embedding_sc",
    )
    return run(user_ids.astype(jnp.int32), item_ids.astype(jnp.int32),
               user_table, item_table, item_bias.reshape(-1))


# speedup vs baseline: 1.1762x; 1.1762x over previous
import jax
import jax.numpy as jnp
from jax import lax
from jax.experimental import pallas as pl
from jax.experimental.pallas import tpu as pltpu
from jax.experimental.pallas import tpu_sc as plsc

NC = 2
NS = 16
L = 16
NW = NC * NS

BATCH = 16384
EMB = 64
B_PER_W = BATCH // NW
PASSES = 2
PR = B_PER_W // PASSES
RB = 16
NRB = B_PER_W // RB
GROUPS = PR // L


def _cf_body(user_ids_hbm, item_ids_hbm, user_table_hbm, item_table_hbm,
             item_bias_hbm, out_hbm,
             uidx_v, iidx_v, urows_v, irows_v, bias_v, out_v, sem):
    wid = lax.axis_index("s") * NC + lax.axis_index("c")
    base = wid * B_PER_W

    pltpu.sync_copy(user_ids_hbm.at[pl.ds(base, B_PER_W)], uidx_v)
    pltpu.sync_copy(item_ids_hbm.at[pl.ds(base, B_PER_W)], iidx_v)

    lanes = lax.iota(jnp.int32, L)

    for p in range(PASSES):
        def row_batch(rb, _):
            r0 = rb * RB
            uvec = uidx_v[pl.ds(p * PR + r0, RB)]
            ivec = iidx_v[pl.ds(p * PR + r0, RB)]
            copies = []
            for j in range(RB):
                u = uvec[j]
                i = ivec[j]
                copies.append(pltpu.async_copy(
                    user_table_hbm.at[pl.ds(u, 1), :],
                    urows_v.at[pl.ds(r0 + j, 1), :], sem))
                copies.append(pltpu.async_copy(
                    item_table_hbm.at[pl.ds(i, 1), :],
                    irows_v.at[pl.ds(r0 + j, 1), :], sem))
                copies.append(pltpu.async_copy(
                    item_bias_hbm.at[pl.ds(i, 1), :],
                    bias_v.at[pl.ds(r0 + j, 1), :], sem))
            for cp in copies:
                cp.wait()
            return 0

        lax.fori_loop(0, PR // RB, row_batch, 0)

        def group(g, _):
            row16 = g * L + lanes
            zeros = jnp.zeros((L,), jnp.int32)
            acc = plsc.load_gather(bias_v, [row16, zeros])
            for j in range(EMB):
                colj = jnp.full((L,), j, jnp.int32)
                u = plsc.load_gather(urows_v, [row16, colj])
                v = plsc.load_gather(irows_v, [row16, colj])
                acc = acc + u * v
            out_v[pl.ds(p * PR + g * L, L)] = acc
            return 0

        lax.fori_loop(0, GROUPS, group, 0)

    pltpu.sync_copy(out_v, out_hbm.at[pl.ds(base, B_PER_W)])


@jax.jit
def kernel(user_ids, item_ids, user_table, item_table, item_bias):
    mesh = plsc.VectorSubcoreMesh(core_axis_name="c", subcore_axis_name="s")
    run = pl.kernel(
        _cf_body,
        out_type=jax.ShapeDtypeStruct((BATCH,), jnp.float32),
        mesh=mesh,
        scratch_types=[
            pltpu.VMEM((B_PER_W,), jnp.int32),
            pltpu.VMEM((B_PER_W,), jnp.int32),
            pltpu.VMEM((PR, EMB), jnp.float32),
            pltpu.VMEM((PR, EMB), jnp.float32),
            pltpu.VMEM((PR, 1), jnp.float32),
            pltpu.VMEM((B_PER_W,), jnp.float32),
            pltpu.SemaphoreType.DMA,
        ],
        compiler_params=pltpu.CompilerParams(needs_layout_passes=False,
                                             use_tc_tiling_on_sc=True),
        name="cf_embedding_sc",
    )
    return run(user_ids.astype(jnp.int32), item_ids.astype(jnp.int32),
               user_table, item_table, item_bias)


# speedup vs baseline: 1.1971x; 1.0178x over previous
import jax
import jax.numpy as jnp
from jax import lax
from jax.experimental import pallas as pl
from jax.experimental.pallas import tpu as pltpu
from jax.experimental.pallas import tpu_sc as plsc

NC = 2
NS = 16
L = 16
NW = NC * NS

BATCH = 16384
EMB = 64
B_PER_W = BATCH // NW
PASSES = 2
PR = B_PER_W // PASSES
RB = 16
NRB = B_PER_W // RB
GROUPS = PR // L


def _cf_body(user_ids_hbm, item_ids_hbm, user_table_hbm, item_table_hbm,
             item_bias_hbm, out_hbm,
             uidx_v, iidx_v, urows_v, irows_v, bias_v, out_v, sem):
    wid = lax.axis_index("s") * NC + lax.axis_index("c")
    base = wid * B_PER_W

    pltpu.sync_copy(user_ids_hbm.at[pl.ds(base, B_PER_W)], uidx_v)
    pltpu.sync_copy(item_ids_hbm.at[pl.ds(base, B_PER_W)], iidx_v)

    lanes = lax.iota(jnp.int32, L)

    for p in range(PASSES):
        def row_batch(rb, _):
            r0 = rb * RB
            uvec = uidx_v[pl.ds(p * PR + r0, RB)]
            ivec = iidx_v[pl.ds(p * PR + r0, RB)]
            for j in range(RB):
                u = uvec[j]
                i = ivec[j]
                pltpu.async_copy(
                    user_table_hbm.at[pl.ds(u, 1), :],
                    urows_v.at[pl.ds(r0 + j, 1), :], sem)
                pltpu.async_copy(
                    item_table_hbm.at[pl.ds(i, 1), :],
                    irows_v.at[pl.ds(r0 + j, 1), :], sem)
                pltpu.async_copy(
                    item_bias_hbm.at[pl.ds(i, 1), :],
                    bias_v.at[pl.ds(r0 + j, 1), :], sem)
            return 0

        lax.fori_loop(0, PR // RB, row_batch, 0)

        def drain_batch(rb, _):
            r0 = rb * RB
            for j in range(RB):
                pltpu.make_async_copy(
                    user_table_hbm.at[pl.ds(0, 1), :],
                    urows_v.at[pl.ds(r0 + j, 1), :], sem).wait()
                pltpu.make_async_copy(
                    item_table_hbm.at[pl.ds(0, 1), :],
                    irows_v.at[pl.ds(r0 + j, 1), :], sem).wait()
                pltpu.make_async_copy(
                    item_bias_hbm.at[pl.ds(0, 1), :],
                    bias_v.at[pl.ds(r0 + j, 1), :], sem).wait()
            return 0

        lax.fori_loop(0, PR // RB, drain_batch, 0)

        def group(g, _):
            row16 = g * L + lanes
            zeros = jnp.zeros((L,), jnp.int32)
            acc = plsc.load_gather(bias_v, [row16, zeros])
            for j in range(EMB):
                colj = jnp.full((L,), j, jnp.int32)
                u = plsc.load_gather(urows_v, [row16, colj])
                v = plsc.load_gather(irows_v, [row16, colj])
                acc = acc + u * v
            out_v[pl.ds(p * PR + g * L, L)] = acc
            return 0

        lax.fori_loop(0, GROUPS, group, 0)

    pltpu.sync_copy(out_v, out_hbm.at[pl.ds(base, B_PER_W)])


@jax.jit
def kernel(user_ids, item_ids, user_table, item_table, item_bias):
    mesh = plsc.VectorSubcoreMesh(core_axis_name="c", subcore_axis_name="s")
    run = pl.kernel(
        _cf_body,
        out_type=jax.ShapeDtypeStruct((BATCH,), jnp.float32),
        mesh=mesh,
        scratch_types=[
            pltpu.VMEM((B_PER_W,), jnp.int32),
            pltpu.VMEM((B_PER_W,), jnp.int32),
            pltpu.VMEM((PR, EMB), jnp.float32),
            pltpu.VMEM((PR, EMB), jnp.float32),
            pltpu.VMEM((PR, 1), jnp.float32),
            pltpu.VMEM((B_PER_W,), jnp.float32),
            pltpu.SemaphoreType.DMA,
        ],
        compiler_params=pltpu.CompilerParams(needs_layout_passes=False,
                                             use_tc_tiling_on_sc=True),
        name="cf_embedding_sc",
    )
    return run(user_ids.astype(jnp.int32), item_ids.astype(jnp.int32),
               user_table, item_table, item_bias)


# speedup vs baseline: 1.1980x; 1.0008x over previous
import jax
import jax.numpy as jnp
from jax import lax
from jax.experimental import pallas as pl
from jax.experimental.pallas import tpu as pltpu
from jax.experimental.pallas import tpu_sc as plsc

NC = 2
NS = 16
L = 16
NW = NC * NS

BATCH = 16384
EMB = 64
B_PER_W = BATCH // NW
PASSES = 2
PR = B_PER_W // PASSES
RB = 16
NRB = B_PER_W // RB
GROUPS = PR // L


def _cf_body(user_ids_hbm, item_ids_hbm, user_table_hbm, item_table_hbm,
             item_bias_hbm, out_hbm,
             uidx_v, iidx_v, urows_v, irows_v, bias_v, out_v, sem):
    wid = lax.axis_index("s") * NC + lax.axis_index("c")
    base = wid * B_PER_W

    pltpu.sync_copy(user_ids_hbm.at[pl.ds(base, B_PER_W)], uidx_v)
    pltpu.sync_copy(item_ids_hbm.at[pl.ds(base, B_PER_W)], iidx_v)

    lanes = lax.iota(jnp.int32, L)

    for p in range(PASSES):
        def row_batch(rb, _):
            r0 = rb * RB
            uvec = uidx_v[pl.ds(p * PR + r0, RB)]
            ivec = iidx_v[pl.ds(p * PR + r0, RB)]
            for j in range(RB):
                u = uvec[j]
                i = ivec[j]
                pltpu.async_copy(
                    user_table_hbm.at[pl.ds(u, 1), :],
                    urows_v.at[pl.ds(r0 + j, 1), :], sem.at[(3 * j) % 8])
                pltpu.async_copy(
                    item_table_hbm.at[pl.ds(i, 1), :],
                    irows_v.at[pl.ds(r0 + j, 1), :], sem.at[(3 * j + 1) % 8])
                pltpu.async_copy(
                    item_bias_hbm.at[pl.ds(i, 1), :],
                    bias_v.at[pl.ds(r0 + j, 1), :], sem.at[(3 * j + 2) % 8])
            return 0

        lax.fori_loop(0, PR // RB, row_batch, 0)

        def drain_batch(rb, _):
            r0 = rb * RB
            for j in range(RB):
                pltpu.make_async_copy(
                    user_table_hbm.at[pl.ds(0, 1), :],
                    urows_v.at[pl.ds(r0 + j, 1), :], sem.at[(3 * j) % 8]).wait()
                pltpu.make_async_copy(
                    item_table_hbm.at[pl.ds(0, 1), :],
                    irows_v.at[pl.ds(r0 + j, 1), :], sem.at[(3 * j + 1) % 8]).wait()
                pltpu.make_async_copy(
                    item_bias_hbm.at[pl.ds(0, 1), :],
                    bias_v.at[pl.ds(r0 + j, 1), :], sem.at[(3 * j + 2) % 8]).wait()
            return 0

        lax.fori_loop(0, PR // RB, drain_batch, 0)

        def group(g, _):
            row16 = g * L + lanes
            zeros = jnp.zeros((L,), jnp.int32)
            acc = plsc.load_gather(bias_v, [row16, zeros])
            for j in range(EMB):
                colj = jnp.full((L,), j, jnp.int32)
                u = plsc.load_gather(urows_v, [row16, colj])
                v = plsc.load_gather(irows_v, [row16, colj])
                acc = acc + u * v
            out_v[pl.ds(p * PR + g * L, L)] = acc
            return 0

        lax.fori_loop(0, GROUPS, group, 0)

    pltpu.sync_copy(out_v, out_hbm.at[pl.ds(base, B_PER_W)])


@jax.jit
def kernel(user_ids, item_ids, user_table, item_table, item_bias):
    mesh = plsc.VectorSubcoreMesh(core_axis_name="c", subcore_axis_name="s")
    run = pl.kernel(
        _cf_body,
        out_type=jax.ShapeDtypeStruct((BATCH,), jnp.float32),
        mesh=mesh,
        scratch_types=[
            pltpu.VMEM((B_PER_W,), jnp.int32),
            pltpu.VMEM((B_PER_W,), jnp.int32),
            pltpu.VMEM((PR, EMB), jnp.float32),
            pltpu.VMEM((PR, EMB), jnp.float32),
            pltpu.VMEM((PR, 1), jnp.float32),
            pltpu.VMEM((B_PER_W,), jnp.float32),
            pltpu.SemaphoreType.DMA((8,)),
        ],
        compiler_params=pltpu.CompilerParams(needs_layout_passes=False,
                                             use_tc_tiling_on_sc=True),
        name="cf_embedding_sc",
    )
    return run(user_ids.astype(jnp.int32), item_ids.astype(jnp.int32),
               user_table, item_table, item_bias)
